# Initial kernel scaffold; baseline (speedup 1.0000x reference)
import jax
import jax.numpy as jnp
from jax.experimental import pallas as pl


def kernel(x, edge_index, edge_weight, self_k0, nbr_k0, b0, self_k1, nbr_k1, b1, mlp_w1, mlp_b1, mlp_w2, mlp_b2):
    raise NotImplementedError("write your pallas kernel here")


# speedup vs baseline: 9.5744x; 9.5744x over previous
import jax
import jax.numpy as jnp
from jax import lax
from jax.experimental import pallas as pl
from jax.experimental.pallas import tpu as pltpu
from jax.experimental.pallas import tpu_sc as plsc

N = 10000
NP = 10240
E = 320000
D = 128
H = 256
C = 6

NCORE = 2
NSUB = 16
CB = 125
RPT = NP // NSUB


def _sc_agg(table, row4, col4, zeros, with_deg, *, chunks):
    mesh = plsc.VectorSubcoreMesh(core_axis_name="c", subcore_axis_name="s")
    out_type = [jax.ShapeDtypeStruct((NCORE, NP, D), jnp.float32)]
    if with_deg:
        out_type.append(
            jax.ShapeDtypeStruct((NCORE * NSUB, 1, NP), jnp.float32))
    STAGE = 80
    assert chunks % STAGE == 0
    phases = chunks // STAGE
    scratch = [
        pltpu.VMEM_SHARED((NP, D), jnp.float32),
        pltpu.VMEM((STAGE, CB), jnp.int32),
        pltpu.VMEM((STAGE, CB), jnp.int32),
        pltpu.VMEM((CB, D), jnp.float32),
        pltpu.SemaphoreType.DMA,
    ]
    if with_deg:
        scratch.append(pltpu.VMEM((NP,), jnp.float32))

    def body(*refs):
        if with_deg:
            (table_h, row_h, col_h, zeros_h, agg_o, deg_o,
             acc, row_v, col_v, gbuf, sem, deg_v) = refs
        else:
            (table_h, row_h, col_h, zeros_h, agg_o,
             acc, row_v, col_v, gbuf, sem) = refs
        c = lax.axis_index("c")
        s = lax.axis_index("s")

        pltpu.sync_copy(zeros_h.at[pl.ds(s * RPT, RPT)],
                        acc.at[pl.ds(s * RPT, RPT)])

        if with_deg:
            zeros16 = jnp.zeros((16,), jnp.float32)

            def dzero(i, carry):
                deg_v[pl.ds(i * 16, 16)] = zeros16
                return carry

            lax.fori_loop(0, NP // 16, dzero, 0)

        plsc.subcore_barrier()

        ones16 = jnp.ones((16,), jnp.float32)
        tailmask = lax.iota(jnp.int32, 16) >= 3

        for p in range(phases):
            pltpu.sync_copy(row_h.at[c, s, pl.ds(p * STAGE, STAGE)], row_v)
            pltpu.sync_copy(col_h.at[c, s, pl.ds(p * STAGE, STAGE)], col_v)

            if with_deg:
                def dacc(j, carry):
                    for k in range(7):
                        idx = row_v[j, pl.ds(k * 16, 16)]
                        plsc.addupdate_scatter(deg_v, [idx], ones16)
                    idx = row_v[j, pl.ds(CB - 16, 16)]
                    plsc.addupdate_scatter(deg_v, [idx], ones16,
                                           mask=tailmask)
                    return carry

                lax.fori_loop(0, STAGE, dacc, 0)

            def chunk(j, carry):
                pltpu.async_copy(table_h.at[col_v.at[j]], gbuf, sem).wait()
                pltpu.sync_copy(gbuf, acc.at[row_v.at[j]], add=True)
                return carry

            lax.fori_loop(0, STAGE, chunk, 0)

        if with_deg:
            w = c * NSUB + s
            pltpu.sync_copy(deg_v, deg_o.at[w, 0])

        plsc.subcore_barrier()
        pltpu.sync_copy(acc.at[pl.ds(s * RPT, RPT)],
                        agg_o.at[c, pl.ds(s * RPT, RPT)])

    return pl.kernel(
        body, out_type=out_type, mesh=mesh, scratch_types=scratch,
        compiler_params=pltpu.CompilerParams(needs_layout_passes=False),
    )(table, row4, col4, zeros)


def _tc1(x, agg0, degp, sk0, nk0, b0, sk1, nk1):
    BM = 1024
    f32 = jnp.float32
    sk1r = sk1.reshape(2, H, H)
    nk1r = nk1.reshape(2, H, H)
    b0r = b0.reshape(1, 2 * H)

    def body(x_r, a_r, degp_r, sk0_r, nk0_r, b0_r, sk1_r, nk1_r, s1_r, t1_r):
        deg = jnp.maximum(jnp.sum(degp_r[...], axis=0), 1.0)
        mean = (a_r[0] + a_r[1]) / deg[:, None]
        h1a = jnp.maximum(
            jnp.dot(x_r[...], sk0_r[...], preferred_element_type=f32)
            + b0_r[0, :H], 0.0)
        h1b = jnp.maximum(
            jnp.dot(mean, nk0_r[...], preferred_element_type=f32)
            + b0_r[0, H:], 0.0)
        s1_r[...] = (jnp.dot(h1a, sk1_r[0], preferred_element_type=f32)
                     + jnp.dot(h1b, sk1_r[1], preferred_element_type=f32))
        t1 = (jnp.dot(h1a, nk1_r[0], preferred_element_type=f32)
              + jnp.dot(h1b, nk1_r[1], preferred_element_type=f32))
        t1_r[0] = t1[:, :D]
        t1_r[1] = t1[:, D:]

    return pl.pallas_call(
        body,
        grid=(NP // BM,),
        in_specs=[
            pl.BlockSpec((BM, D), lambda i: (i, 0)),
            pl.BlockSpec((2, BM, D), lambda i: (0, i, 0)),
            pl.BlockSpec((NCORE * NSUB, BM), lambda i: (0, i)),
            pl.BlockSpec((D, H), lambda i: (0, 0)),
            pl.BlockSpec((D, H), lambda i: (0, 0)),
            pl.BlockSpec((1, 2 * H), lambda i: (0, 0)),
            pl.BlockSpec((2, H, H), lambda i: (0, 0, 0)),
            pl.BlockSpec((2, H, H), lambda i: (0, 0, 0)),
        ],
        out_specs=[
            pl.BlockSpec((BM, H), lambda i: (i, 0)),
            pl.BlockSpec((2, BM, D), lambda i: (0, i, 0)),
        ],
        out_shape=[
            jax.ShapeDtypeStruct((NP, H), jnp.float32),
            jax.ShapeDtypeStruct((2, NP, D), jnp.float32),
        ],
    )(x, agg0, degp, sk0, nk0, b0r, sk1r, nk1r)


def _tc2(s1, agg1, degp, b1, mlp_w1, mlp_b1, mlp_w2, mlp_b2):
    BM = 1024
    f32 = jnp.float32
    b1r = b1.reshape(1, 2 * H)
    w1r = mlp_w1.reshape(2, H, H)
    b1mr = mlp_b1.reshape(1, H)
    w2p = jnp.pad(mlp_w2, ((0, 0), (0, 128 - C)))
    b2mr = jnp.pad(mlp_b2, (0, 128 - C)).reshape(1, 128)

    def body(s1_r, a_r, degp_r, b1_r, w1_r, b1m_r, w2_r, b2m_r, o_r):
        deg = jnp.maximum(jnp.sum(degp_r[...], axis=0), 1.0)
        m = jnp.concatenate([a_r[0], a_r[1]], axis=1) / deg[:, None]
        h2a = jnp.maximum(s1_r[...] + b1_r[0, :H], 0.0)
        h2b = jnp.maximum(m + b1_r[0, H:], 0.0)
        h3 = jnp.maximum(
            jnp.dot(h2a, w1_r[0], preferred_element_type=f32)
            + jnp.dot(h2b, w1_r[1], preferred_element_type=f32)
            + b1m_r[0], 0.0)
        o_r[...] = jnp.dot(h3, w2_r[...], preferred_element_type=f32) + b2m_r[0]

    return pl.pallas_call(
        body,
        grid=(NP // BM,),
        in_specs=[
            pl.BlockSpec((BM, H), lambda i: (i, 0)),
            pl.BlockSpec((2, BM, D), lambda i: (0, i, 0)),
            pl.BlockSpec((NCORE * NSUB, BM), lambda i: (0, i)),
            pl.BlockSpec((1, 2 * H), lambda i: (0, 0)),
            pl.BlockSpec((2, H, H), lambda i: (0, 0, 0)),
            pl.BlockSpec((1, H), lambda i: (0, 0)),
            pl.BlockSpec((H, 128), lambda i: (0, 0)),
            pl.BlockSpec((1, 128), lambda i: (0, 0)),
        ],
        out_specs=pl.BlockSpec((BM, 128), lambda i: (i, 0)),
        out_shape=jax.ShapeDtypeStruct((NP, 128), jnp.float32),
    )(s1, agg1, degp, b1r, w1r, b1mr, w2p, b2mr)


def kernel(x, edge_index, edge_weight, self_k0, nbr_k0, b0,
           self_k1, nbr_k1, b1, mlp_w1, mlp_b1, mlp_w2, mlp_b2):
    row = edge_index[0]
    col = edge_index[1]
    xp = jnp.pad(x, ((0, NP - N), (0, 0)))
    zeros = jnp.zeros((NP, D), jnp.float32)

    ch_a = E // (NCORE * NSUB * CB)
    row4a = row.reshape(NCORE, NSUB, ch_a, CB)
    col4a = col.reshape(NCORE, NSUB, ch_a, CB)
    agg0, degp = _sc_agg(xp, row4a, col4a, zeros, True, chunks=ch_a)
    degp = degp.reshape(NCORE * NSUB, NP)

    s1, t1 = _tc1(xp, agg0, degp, self_k0, nbr_k0, b0, self_k1, nbr_k1)

    ch_b = E // (NSUB * CB)
    rowb = jnp.tile(row.reshape(1, NSUB, ch_b, CB), (NCORE, 1, 1, 1))
    colb = jnp.stack([col, col + NP]).reshape(NCORE, NSUB, ch_b, CB)
    (agg1,) = _sc_agg(t1.reshape(2 * NP, D), rowb, colb, zeros,
                      False, chunks=ch_b)

    out = _tc2(s1, agg1, degp, b1, mlp_w1, mlp_b1, mlp_w2, mlp_b2)
    return out[:N, :C]


# speedup vs baseline: 13.5109x; 1.4111x over previous
import jax
import jax.numpy as jnp
from jax import lax
from jax.experimental import pallas as pl
from jax.experimental.pallas import tpu as pltpu
from jax.experimental.pallas import tpu_sc as plsc

N = 10000
NP = 10240
E = 320000
D = 128
H = 256
C = 6

NCORE = 2
NSUB = 16
CB = 125
RPT = NP // NSUB


def _sc_agg(table, row4, col4, zeros, with_deg, *, chunks):
    mesh = plsc.VectorSubcoreMesh(core_axis_name="c", subcore_axis_name="s")
    out_type = [jax.ShapeDtypeStruct((NCORE, NP, D), jnp.float32)]
    if with_deg:
        out_type.append(
            jax.ShapeDtypeStruct((NCORE * NSUB, 1, NP), jnp.float32))
    STAGE = 16
    assert chunks % STAGE == 0 and STAGE % 2 == 0
    phases = chunks // STAGE
    scratch = [
        pltpu.VMEM_SHARED((NP, D), jnp.float32),
        pltpu.VMEM((STAGE, CB), jnp.int32),
        pltpu.VMEM((STAGE, CB), jnp.int32),
        pltpu.VMEM((CB, D), jnp.float32),
        pltpu.VMEM((CB, D), jnp.float32),
        pltpu.SemaphoreType.DMA,
        pltpu.SemaphoreType.DMA,
    ]
    if with_deg:
        scratch.append(pltpu.VMEM((NP,), jnp.float32))

    def body(*refs):
        if with_deg:
            (table_h, row_h, col_h, zeros_h, agg_o, deg_o,
             acc, row_v, col_v, gbuf0, gbuf1, sem0, sem1, deg_v) = refs
        else:
            (table_h, row_h, col_h, zeros_h, agg_o,
             acc, row_v, col_v, gbuf0, gbuf1, sem0, sem1) = refs
        c = lax.axis_index("c")
        s = lax.axis_index("s")

        pltpu.sync_copy(zeros_h.at[pl.ds(s * RPT, RPT)],
                        acc.at[pl.ds(s * RPT, RPT)])

        if with_deg:
            zeros16 = jnp.zeros((16,), jnp.float32)

            def dzero(i, carry):
                deg_v[pl.ds(i * 16, 16)] = zeros16
                return carry

            lax.fori_loop(0, NP // 16, dzero, 0)

        plsc.subcore_barrier()

        ones16 = jnp.ones((16,), jnp.float32)
        tailmask = lax.iota(jnp.int32, 16) >= 3

        def dupd(j):
            if with_deg:
                for k in range(7):
                    idx = row_v[j, pl.ds(k * 16, 16)]
                    plsc.addupdate_scatter(deg_v, [idx], ones16)
                idx = row_v[j, pl.ds(CB - 16, 16)]
                plsc.addupdate_scatter(deg_v, [idx], ones16, mask=tailmask)

        for p in range(phases):
            pltpu.sync_copy(row_h.at[c, s, pl.ds(p * STAGE, STAGE)], row_v)
            pltpu.sync_copy(col_h.at[c, s, pl.ds(p * STAGE, STAGE)], col_v)

            pltpu.async_copy(table_h.at[col_v.at[0]], gbuf0, sem0)

            def pair(i, carry):
                j0 = 2 * i
                pltpu.async_copy(table_h.at[col_v.at[j0 + 1]], gbuf1, sem1)
                pltpu.make_async_copy(
                    table_h.at[col_v.at[j0]], gbuf0, sem0).wait()
                dupd(j0)
                pltpu.sync_copy(gbuf0, acc.at[row_v.at[j0]], add=True)

                @pl.when(i < STAGE // 2 - 1)
                def _():
                    pltpu.async_copy(
                        table_h.at[col_v.at[j0 + 2]], gbuf0, sem0)

                pltpu.make_async_copy(
                    table_h.at[col_v.at[j0 + 1]], gbuf1, sem1).wait()
                dupd(j0 + 1)
                pltpu.sync_copy(gbuf1, acc.at[row_v.at[j0 + 1]], add=True)
                return carry

            lax.fori_loop(0, STAGE // 2, pair, 0)

        if with_deg:
            w = c * NSUB + s
            pltpu.sync_copy(deg_v, deg_o.at[w, 0])

        plsc.subcore_barrier()
        pltpu.sync_copy(acc.at[pl.ds(s * RPT, RPT)],
                        agg_o.at[c, pl.ds(s * RPT, RPT)])

    return pl.kernel(
        body, out_type=out_type, mesh=mesh, scratch_types=scratch,
        compiler_params=pltpu.CompilerParams(needs_layout_passes=False),
    )(table, row4, col4, zeros)


def _tc1(x, agg0, degp, sk0, nk0, b0, sk1, nk1):
    BM = 1024
    f32 = jnp.float32
    sk1r = sk1.reshape(2, H, H)
    nk1r = nk1.reshape(2, H, H)
    b0r = b0.reshape(1, 2 * H)

    def body(x_r, a_r, degp_r, sk0_r, nk0_r, b0_r, sk1_r, nk1_r, s1_r, t1_r):
        deg = jnp.maximum(jnp.sum(degp_r[...], axis=0), 1.0)
        mean = (a_r[0] + a_r[1]) / deg[:, None]
        h1a = jnp.maximum(
            jnp.dot(x_r[...], sk0_r[...], preferred_element_type=f32)
            + b0_r[0, :H], 0.0)
        h1b = jnp.maximum(
            jnp.dot(mean, nk0_r[...], preferred_element_type=f32)
            + b0_r[0, H:], 0.0)
        s1_r[...] = (jnp.dot(h1a, sk1_r[0], preferred_element_type=f32)
                     + jnp.dot(h1b, sk1_r[1], preferred_element_type=f32))
        t1 = (jnp.dot(h1a, nk1_r[0], preferred_element_type=f32)
              + jnp.dot(h1b, nk1_r[1], preferred_element_type=f32))
        t1_r[0] = t1[:, :D]
        t1_r[1] = t1[:, D:]

    return pl.pallas_call(
        body,
        grid=(NP // BM,),
        in_specs=[
            pl.BlockSpec((BM, D), lambda i: (i, 0)),
            pl.BlockSpec((2, BM, D), lambda i: (0, i, 0)),
            pl.BlockSpec((NCORE * NSUB, BM), lambda i: (0, i)),
            pl.BlockSpec((D, H), lambda i: (0, 0)),
            pl.BlockSpec((D, H), lambda i: (0, 0)),
            pl.BlockSpec((1, 2 * H), lambda i: (0, 0)),
            pl.BlockSpec((2, H, H), lambda i: (0, 0, 0)),
            pl.BlockSpec((2, H, H), lambda i: (0, 0, 0)),
        ],
        out_specs=[
            pl.BlockSpec((BM, H), lambda i: (i, 0)),
            pl.BlockSpec((2, BM, D), lambda i: (0, i, 0)),
        ],
        out_shape=[
            jax.ShapeDtypeStruct((NP, H), jnp.float32),
            jax.ShapeDtypeStruct((2, NP, D), jnp.float32),
        ],
    )(x, agg0, degp, sk0, nk0, b0r, sk1r, nk1r)


def _tc2(s1, agg1, degp, b1, mlp_w1, mlp_b1, mlp_w2, mlp_b2):
    BM = 1024
    f32 = jnp.float32
    b1r = b1.reshape(1, 2 * H)
    w1r = mlp_w1.reshape(2, H, H)
    b1mr = mlp_b1.reshape(1, H)
    w2p = jnp.pad(mlp_w2, ((0, 0), (0, 128 - C)))
    b2mr = jnp.pad(mlp_b2, (0, 128 - C)).reshape(1, 128)

    def body(s1_r, a_r, degp_r, b1_r, w1_r, b1m_r, w2_r, b2m_r, o_r):
        deg = jnp.maximum(jnp.sum(degp_r[...], axis=0), 1.0)
        m = jnp.concatenate([a_r[0], a_r[1]], axis=1) / deg[:, None]
        h2a = jnp.maximum(s1_r[...] + b1_r[0, :H], 0.0)
        h2b = jnp.maximum(m + b1_r[0, H:], 0.0)
        h3 = jnp.maximum(
            jnp.dot(h2a, w1_r[0], preferred_element_type=f32)
            + jnp.dot(h2b, w1_r[1], preferred_element_type=f32)
            + b1m_r[0], 0.0)
        o_r[...] = jnp.dot(h3, w2_r[...], preferred_element_type=f32) + b2m_r[0]

    return pl.pallas_call(
        body,
        grid=(NP // BM,),
        in_specs=[
            pl.BlockSpec((BM, H), lambda i: (i, 0)),
            pl.BlockSpec((2, BM, D), lambda i: (0, i, 0)),
            pl.BlockSpec((NCORE * NSUB, BM), lambda i: (0, i)),
            pl.BlockSpec((1, 2 * H), lambda i: (0, 0)),
            pl.BlockSpec((2, H, H), lambda i: (0, 0, 0)),
            pl.BlockSpec((1, H), lambda i: (0, 0)),
            pl.BlockSpec((H, 128), lambda i: (0, 0)),
            pl.BlockSpec((1, 128), lambda i: (0, 0)),
        ],
        out_specs=pl.BlockSpec((BM, 128), lambda i: (i, 0)),
        out_shape=jax.ShapeDtypeStruct((NP, 128), jnp.float32),
    )(s1, agg1, degp, b1r, w1r, b1mr, w2p, b2mr)


def kernel(x, edge_index, edge_weight, self_k0, nbr_k0, b0,
           self_k1, nbr_k1, b1, mlp_w1, mlp_b1, mlp_w2, mlp_b2):
    row = edge_index[0]
    col = edge_index[1]
    xp = jnp.pad(x, ((0, NP - N), (0, 0)))
    zeros = jnp.zeros((NP, D), jnp.float32)

    ch_a = E // (NCORE * NSUB * CB)
    row4a = row.reshape(NCORE, NSUB, ch_a, CB)
    col4a = col.reshape(NCORE, NSUB, ch_a, CB)
    agg0, degp = _sc_agg(xp, row4a, col4a, zeros, True, chunks=ch_a)
    degp = degp.reshape(NCORE * NSUB, NP)

    s1, t1 = _tc1(xp, agg0, degp, self_k0, nbr_k0, b0, self_k1, nbr_k1)

    ch_b = E // (NSUB * CB)
    rowb = jnp.tile(row.reshape(1, NSUB, ch_b, CB), (NCORE, 1, 1, 1))
    colb = jnp.stack([col, col + NP]).reshape(NCORE, NSUB, ch_b, CB)
    (agg1,) = _sc_agg(t1.reshape(2 * NP, D), rowb, colb, zeros,
                      False, chunks=ch_b)

    out = _tc2(s1, agg1, degp, b1, mlp_w1, mlp_b1, mlp_w2, mlp_b2)
    return out[:N, :C]


# speedup vs baseline: 13.8267x; 1.0234x over previous
import jax
import jax.numpy as jnp
from jax import lax
from jax.experimental import pallas as pl
from jax.experimental.pallas import tpu as pltpu
from jax.experimental.pallas import tpu_sc as plsc

N = 10000
NP = 10240
E = 320000
D = 128
H = 256
C = 6

NCORE = 2
NSUB = 16
CB = 125
RPT = NP // NSUB
BM = 1024
GRID = NP // BM


def _sc_agg(table, ei, zeros, with_deg, *, chunks, split_edges):
    mesh = plsc.VectorSubcoreMesh(core_axis_name="c", subcore_axis_name="s")
    out_type = [jax.ShapeDtypeStruct((NCORE, NP, D), jnp.float32)]
    if with_deg:
        out_type.append(
            jax.ShapeDtypeStruct((NCORE * NSUB, 1, NP), jnp.float32))
    STAGE = 16
    assert chunks % STAGE == 0
    phases = chunks // STAGE
    scratch = [
        pltpu.VMEM_SHARED((NP, D), jnp.float32),
        pltpu.VMEM((STAGE, CB), jnp.int32),
        pltpu.VMEM((STAGE, CB), jnp.int32),
        pltpu.VMEM((CB, D), jnp.float32),
        pltpu.VMEM((CB, D), jnp.float32),
        pltpu.SemaphoreType.DMA,
        pltpu.SemaphoreType.DMA,
    ]
    if with_deg:
        scratch.append(pltpu.VMEM((NP,), jnp.float32))

    def body(*refs):
        if with_deg:
            (table_h, ei_h, zeros_h, agg_o, deg_o,
             acc, row_v, col_v, gbuf0, gbuf1, sem0, sem1, deg_v) = refs
        else:
            (table_h, ei_h, zeros_h, agg_o,
             acc, row_v, col_v, gbuf0, gbuf1, sem0, sem1) = refs
        c = lax.axis_index("c")
        s = lax.axis_index("s")
        g = c if split_edges else 0
        tbl = table_h.at[c] if table.ndim == 3 else table_h

        pltpu.sync_copy(zeros_h.at[pl.ds(s * RPT, RPT)],
                        acc.at[pl.ds(s * RPT, RPT)])

        if with_deg:
            zeros16 = jnp.zeros((16,), jnp.float32)

            def dzero(i, carry):
                deg_v[pl.ds(i * 16, 16)] = zeros16
                return carry

            lax.fori_loop(0, NP // 16, dzero, 0)

        plsc.subcore_barrier()

        ones16 = jnp.ones((16,), jnp.float32)
        tailmask = lax.iota(jnp.int32, 16) >= 3

        def dupd(j):
            if with_deg:
                for k in range(7):
                    idx = row_v[j, pl.ds(k * 16, 16)]
                    plsc.addupdate_scatter(deg_v, [idx], ones16)
                idx = row_v[j, pl.ds(CB - 16, 16)]
                plsc.addupdate_scatter(deg_v, [idx], ones16, mask=tailmask)

        for p in range(phases):
            pltpu.sync_copy(ei_h.at[0, g, s, pl.ds(p * STAGE, STAGE)], row_v)
            pltpu.sync_copy(ei_h.at[1, g, s, pl.ds(p * STAGE, STAGE)], col_v)

            pltpu.async_copy(tbl.at[col_v.at[0]], gbuf0, sem0)

            def pair(i, carry):
                j0 = 2 * i
                pltpu.async_copy(tbl.at[col_v.at[j0 + 1]], gbuf1, sem1)
                pltpu.make_async_copy(
                    tbl.at[col_v.at[j0]], gbuf0, sem0).wait()
                dupd(j0)
                pltpu.sync_copy(gbuf0, acc.at[row_v.at[j0]], add=True)

                @pl.when(i < STAGE // 2 - 1)
                def _():
                    pltpu.async_copy(tbl.at[col_v.at[j0 + 2]], gbuf0, sem0)

                pltpu.make_async_copy(
                    tbl.at[col_v.at[j0 + 1]], gbuf1, sem1).wait()
                dupd(j0 + 1)
                pltpu.sync_copy(gbuf1, acc.at[row_v.at[j0 + 1]], add=True)
                return carry

            lax.fori_loop(0, STAGE // 2, pair, 0)

        if with_deg:
            w = c * NSUB + s
            pltpu.sync_copy(deg_v, deg_o.at[w, 0])

        plsc.subcore_barrier()
        pltpu.sync_copy(acc.at[pl.ds(s * RPT, RPT)],
                        agg_o.at[c, pl.ds(s * RPT, RPT)])

    return pl.kernel(
        body, out_type=out_type, mesh=mesh, scratch_types=scratch,
        compiler_params=pltpu.CompilerParams(needs_layout_passes=False),
    )(table, ei, zeros)


_f32 = jnp.float32


def _dot(a, b):
    return jnp.dot(a, b, preferred_element_type=_f32)


def _wspec(*shape):
    nd = len(shape)
    return pl.BlockSpec(shape, lambda i, nd=nd: (0,) * nd)


def _rspec(*shape):
    return pl.BlockSpec(shape, lambda i: (i,) + (0,) * (len(shape) - 1))


def _tc_pre1(x, sk0, b0r, sk1r, nk1r):
    def body(x_r, sk0_r, b0_r, sk1_r, nk1_r, s1a_r, t1a_r):
        h1a = jnp.maximum(_dot(x_r[...], sk0_r[...]) + b0_r[0, :H], 0.0)
        s1a_r[...] = _dot(h1a, sk1_r[0])
        t1a = _dot(h1a, nk1_r[0])
        t1a_r[0] = t1a[:, :D]
        t1a_r[1] = t1a[:, D:]

    return pl.pallas_call(
        body,
        grid=(GRID,),
        in_specs=[_rspec(BM, D), _wspec(D, H), _wspec(1, 2 * H),
                  _wspec(2, H, H), _wspec(2, H, H)],
        out_specs=[_rspec(BM, H), pl.BlockSpec((2, BM, D), lambda i: (0, i, 0))],
        out_shape=[jax.ShapeDtypeStruct((NP, H), _f32),
                   jax.ShapeDtypeStruct((2, NP, D), _f32)],
    )(x, sk0, b0r, sk1r, nk1r)


def _tc_mid(agg0, degp, s1a, t1a, nk0, b0r, sk1r, nk1r):
    def body(a_r, degp_r, s1a_r, t1a_r, nk0_r, b0_r, sk1_r, nk1_r,
             s1_r, t1_r):
        deg = jnp.maximum(jnp.sum(degp_r[...], axis=0), 1.0)
        mean = (a_r[0] + a_r[1]) / deg[:, None]
        h1b = jnp.maximum(_dot(mean, nk0_r[...]) + b0_r[0, H:], 0.0)
        s1_r[...] = s1a_r[...] + _dot(h1b, sk1_r[1])
        t1b = _dot(h1b, nk1_r[1])
        t1_r[0] = t1a_r[0] + t1b[:, :D]
        t1_r[1] = t1a_r[1] + t1b[:, D:]

    return pl.pallas_call(
        body,
        grid=(GRID,),
        in_specs=[pl.BlockSpec((2, BM, D), lambda i: (0, i, 0)),
                  pl.BlockSpec((NCORE * NSUB, BM), lambda i: (0, i)),
                  _rspec(BM, H),
                  pl.BlockSpec((2, BM, D), lambda i: (0, i, 0)),
                  _wspec(D, H), _wspec(1, 2 * H),
                  _wspec(2, H, H), _wspec(2, H, H)],
        out_specs=[_rspec(BM, H), pl.BlockSpec((2, BM, D), lambda i: (0, i, 0))],
        out_shape=[jax.ShapeDtypeStruct((NP, H), _f32),
                   jax.ShapeDtypeStruct((2, NP, D), _f32)],
    )(agg0, degp, s1a, t1a, nk0, b0r, sk1r, nk1r)


def _tc_pre2(s1, b1r, w1r):
    def body(s1_r, b1_r, w1_r, p3_r):
        h2a = jnp.maximum(s1_r[...] + b1_r[0, :H], 0.0)
        p3_r[...] = _dot(h2a, w1_r[0])

    return pl.pallas_call(
        body,
        grid=(GRID,),
        in_specs=[_rspec(BM, H), _wspec(1, 2 * H), _wspec(2, H, H)],
        out_specs=_rspec(BM, H),
        out_shape=jax.ShapeDtypeStruct((NP, H), _f32),
    )(s1, b1r, w1r)


def _tc_post(agg1, degp, p3, b1r, w1r, b1mr, w2p, b2mr):
    def body(a_r, degp_r, p3_r, b1_r, w1_r, b1m_r, w2_r, b2m_r, o_r):
        deg = jnp.maximum(jnp.sum(degp_r[...], axis=0), 1.0)
        m = jnp.concatenate([a_r[0], a_r[1]], axis=1) / deg[:, None]
        h2b = jnp.maximum(m + b1_r[0, H:], 0.0)
        h3 = jnp.maximum(p3_r[...] + _dot(h2b, w1_r[1]) + b1m_r[0], 0.0)
        o_r[...] = _dot(h3, w2_r[...]) + b2m_r[0]

    return pl.pallas_call(
        body,
        grid=(GRID,),
        in_specs=[pl.BlockSpec((2, BM, D), lambda i: (0, i, 0)),
                  pl.BlockSpec((NCORE * NSUB, BM), lambda i: (0, i)),
                  _rspec(BM, H), _wspec(1, 2 * H), _wspec(2, H, H),
                  _wspec(1, H), _wspec(H, 128), _wspec(1, 128)],
        out_specs=_rspec(BM, 128),
        out_shape=jax.ShapeDtypeStruct((NP, 128), _f32),
    )(agg1, degp, p3, b1r, w1r, b1mr, w2p, b2mr)


def kernel(x, edge_index, edge_weight, self_k0, nbr_k0, b0,
           self_k1, nbr_k1, b1, mlp_w1, mlp_b1, mlp_w2, mlp_b2):
    xp = jnp.pad(x, ((0, NP - N), (0, 0)))
    zeros = jnp.zeros((NP, D), jnp.float32)
    b0r = b0.reshape(1, 2 * H)
    b1r = b1.reshape(1, 2 * H)
    sk1r = self_k1.reshape(2, H, H)
    nk1r = nbr_k1.reshape(2, H, H)
    w1r = mlp_w1.reshape(2, H, H)
    b1mr = mlp_b1.reshape(1, H)
    w2p = jnp.pad(mlp_w2, ((0, 0), (0, 128 - C)))
    b2mr = jnp.pad(mlp_b2, (0, 128 - C)).reshape(1, 128)

    ch_a = E // (NCORE * NSUB * CB)
    ei_a = edge_index.reshape(2, NCORE, NSUB, ch_a, CB)
    agg0, degp = _sc_agg(xp, ei_a, zeros, True, chunks=ch_a,
                         split_edges=True)
    degp = degp.reshape(NCORE * NSUB, NP)
    s1a, t1a = _tc_pre1(xp, self_k0, b0r, sk1r, nk1r)

    s1, t1 = _tc_mid(agg0, degp, s1a, t1a, nbr_k0, b0r, sk1r, nk1r)

    ch_b = E // (NSUB * CB)
    ei_b = edge_index.reshape(2, 1, NSUB, ch_b, CB)
    (agg1,) = _sc_agg(t1, ei_b, zeros, False, chunks=ch_b,
                      split_edges=False)
    p3 = _tc_pre2(s1, b1r, w1r)

    out = _tc_post(agg1, degp, p3, b1r, w1r, b1mr, w2p, b2mr)
    return out[:N, :C]


# speedup vs baseline: 14.0692x; 1.0175x over previous
import jax
import jax.numpy as jnp
from jax import lax
from jax.experimental import pallas as pl
from jax.experimental.pallas import tpu as pltpu
from jax.experimental.pallas import tpu_sc as plsc

N = 10000
NP = 10240
E = 320000
D = 128
H = 256
C = 6

NCORE = 2
NSUB = 16
CB = 125
NW = NCORE * NSUB
CPW = E // (NW * CB)
RPT = NP // NSUB
BM = 1024
GRID = NP // BM
STAGE = 16


def _sc_agg(table, ei4, zeros, with_deg, *, split_edges):
    mesh = plsc.VectorSubcoreMesh(core_axis_name="c", subcore_axis_name="s")
    out_type = [jax.ShapeDtypeStruct((NCORE, NP, D), jnp.float32)]
    if with_deg:
        out_type.append(jax.ShapeDtypeStruct((NW, 1, NP), jnp.float32))
    spw = CPW // STAGE
    phases = spw if split_edges else 2 * spw
    scratch = [
        pltpu.VMEM_SHARED((NP, D), jnp.float32),
        pltpu.VMEM((STAGE, CB), jnp.int32),
        pltpu.VMEM((STAGE, CB), jnp.int32),
        pltpu.VMEM((CB, D), jnp.float32),
        pltpu.VMEM((CB, D), jnp.float32),
        pltpu.SemaphoreType.DMA,
        pltpu.SemaphoreType.DMA,
    ]
    if with_deg:
        scratch.append(pltpu.VMEM((NP,), jnp.float32))

    def body(*refs):
        if with_deg:
            (table_h, ei_h, zeros_h, agg_o, deg_o,
             acc, row_v, col_v, gbuf0, gbuf1, sem0, sem1, deg_v) = refs
        else:
            (table_h, ei_h, zeros_h, agg_o,
             acc, row_v, col_v, gbuf0, gbuf1, sem0, sem1) = refs
        c = lax.axis_index("c")
        s = lax.axis_index("s")
        tbl = table_h.at[c] if table.ndim == 3 else table_h

        pltpu.sync_copy(zeros_h.at[pl.ds(s * RPT, RPT)],
                        acc.at[pl.ds(s * RPT, RPT)])

        if with_deg:
            zeros16 = jnp.zeros((16,), jnp.float32)

            def dzero(i, carry):
                deg_v[pl.ds(i * 16, 16)] = zeros16
                return carry

            lax.fori_loop(0, NP // 16, dzero, 0)

        plsc.subcore_barrier()

        ones16 = jnp.ones((16,), jnp.float32)
        tailmask = lax.iota(jnp.int32, 16) >= 3

        def dupd(j):
            if with_deg:
                for k in range(7):
                    idx = row_v[j, pl.ds(k * 16, 16)]
                    plsc.addupdate_scatter(deg_v, [idx], ones16)
                idx = row_v[j, pl.ds(CB - 16, 16)]
                plsc.addupdate_scatter(deg_v, [idx], ones16, mask=tailmask)

        for p in range(phases):
            if split_edges:
                wrow = c * NSUB + s
                roff = p * STAGE
            else:
                wrow = 2 * s + p // spw
                roff = (p % spw) * STAGE
            pltpu.sync_copy(ei_h.at[0, wrow, pl.ds(roff, STAGE)], row_v)
            pltpu.sync_copy(ei_h.at[1, wrow, pl.ds(roff, STAGE)], col_v)

            pltpu.async_copy(tbl.at[col_v.at[0]], gbuf0, sem0)

            def pair(i, carry):
                j0 = 2 * i
                pltpu.async_copy(tbl.at[col_v.at[j0 + 1]], gbuf1, sem1)
                pltpu.make_async_copy(
                    tbl.at[col_v.at[j0]], gbuf0, sem0).wait()
                dupd(j0)
                pltpu.sync_copy(gbuf0, acc.at[row_v.at[j0]], add=True)

                @pl.when(i < STAGE // 2 - 1)
                def _():
                    pltpu.async_copy(tbl.at[col_v.at[j0 + 2]], gbuf0, sem0)

                pltpu.make_async_copy(
                    tbl.at[col_v.at[j0 + 1]], gbuf1, sem1).wait()
                dupd(j0 + 1)
                pltpu.sync_copy(gbuf1, acc.at[row_v.at[j0 + 1]], add=True)
                return carry

            lax.fori_loop(0, STAGE // 2, pair, 0)

        if with_deg:
            w = c * NSUB + s
            pltpu.sync_copy(deg_v, deg_o.at[w, 0])

        plsc.subcore_barrier()
        pltpu.sync_copy(acc.at[pl.ds(s * RPT, RPT)],
                        agg_o.at[c, pl.ds(s * RPT, RPT)])

    return pl.kernel(
        body, out_type=out_type, mesh=mesh, scratch_types=scratch,
        compiler_params=pltpu.CompilerParams(needs_layout_passes=False),
    )(table, ei4, zeros)


_f32 = jnp.float32


def _dot(a, b):
    return jnp.dot(a, b, preferred_element_type=_f32)


def _wspec(*shape):
    nd = len(shape)
    return pl.BlockSpec(shape, lambda i, nd=nd: (0,) * nd)


def _rspec(*shape):
    return pl.BlockSpec(shape, lambda i: (i,) + (0,) * (len(shape) - 1))


_spec2 = pl.BlockSpec((2, BM, D), lambda i: (0, i, 0))
_dspec = pl.BlockSpec((NW, BM), lambda i: (0, i))


def _tc_pre1(x, sk0, b0r, sk1r, nk1r):
    def body(x_r, sk0_r, b0_r, sk1_r, nk1_r, s1a_r, t1a_r):
        h1a = jnp.maximum(_dot(x_r[...], sk0_r[...]) + b0_r[0, :H], 0.0)
        s1a_r[...] = _dot(h1a, sk1_r[0])
        t1a = _dot(h1a, nk1_r[0])
        t1a_r[0] = t1a[:, :D]
        t1a_r[1] = t1a[:, D:]

    return pl.pallas_call(
        body,
        grid=(GRID,),
        in_specs=[_rspec(BM, D), _wspec(D, H), _wspec(1, 2 * H),
                  _wspec(2, H, H), _wspec(2, H, H)],
        out_specs=[_rspec(BM, H), _spec2],
        out_shape=[jax.ShapeDtypeStruct((NP, H), _f32),
                   jax.ShapeDtypeStruct((2, NP, D), _f32)],
    )(x, sk0, b0r, sk1r, nk1r)


def _tc_mid(agg0, degp, t1a, nk0, b0r, nk1r):
    def body(a_r, degp_r, t1a_r, nk0_r, b0_r, nk1_r, t1_r, h1b_r):
        deg = jnp.maximum(jnp.sum(degp_r[...], axis=0), 1.0)
        mean = (a_r[0] + a_r[1]) / deg[:, None]
        h1b = jnp.maximum(_dot(mean, nk0_r[...]) + b0_r[0, H:], 0.0)
        h1b_r[...] = h1b
        t1b = _dot(h1b, nk1_r[1])
        t1_r[0] = t1a_r[0] + t1b[:, :D]
        t1_r[1] = t1a_r[1] + t1b[:, D:]

    return pl.pallas_call(
        body,
        grid=(GRID,),
        in_specs=[_spec2, _dspec, _spec2,
                  _wspec(D, H), _wspec(1, 2 * H), _wspec(2, H, H)],
        out_specs=[_spec2, _rspec(BM, H)],
        out_shape=[jax.ShapeDtypeStruct((2, NP, D), _f32),
                   jax.ShapeDtypeStruct((NP, H), _f32)],
    )(agg0, degp, t1a, nk0, b0r, nk1r)


def _tc_pre2(s1a, h1b, sk1r, b1r, w1r):
    def body(s1a_r, h1b_r, sk1_r, b1_r, w1_r, p3_r):
        s1 = s1a_r[...] + _dot(h1b_r[...], sk1_r[1])
        h2a = jnp.maximum(s1 + b1_r[0, :H], 0.0)
        p3_r[...] = _dot(h2a, w1_r[0])

    return pl.pallas_call(
        body,
        grid=(GRID,),
        in_specs=[_rspec(BM, H), _rspec(BM, H), _wspec(2, H, H),
                  _wspec(1, 2 * H), _wspec(2, H, H)],
        out_specs=_rspec(BM, H),
        out_shape=jax.ShapeDtypeStruct((NP, H), _f32),
    )(s1a, h1b, sk1r, b1r, w1r)


def _tc_post(agg1, degp, p3, b1r, w1r, b1mr, w2, b2mr):
    def body(a_r, degp_r, p3_r, b1_r, w1_r, b1m_r, w2_r, b2m_r, o_r):
        deg = jnp.maximum(jnp.sum(degp_r[...], axis=0), 1.0)
        m = jnp.concatenate([a_r[0], a_r[1]], axis=1) / deg[:, None]
        h2b = jnp.maximum(m + b1_r[0, H:], 0.0)
        h3 = jnp.maximum(p3_r[...] + _dot(h2b, w1_r[1]) + b1m_r[0], 0.0)
        o_r[...] = _dot(h3, w2_r[...]) + b2m_r[0]

    return pl.pallas_call(
        body,
        grid=(GRID,),
        in_specs=[_spec2, _dspec,
                  _rspec(BM, H), _wspec(1, 2 * H), _wspec(2, H, H),
                  _wspec(1, H), _wspec(H, C), _wspec(1, C)],
        out_specs=_rspec(BM, C),
        out_shape=jax.ShapeDtypeStruct((N, C), _f32),
    )(agg1, degp, p3, b1r, w1r, b1mr, w2, b2mr)


def kernel(x, edge_index, edge_weight, self_k0, nbr_k0, b0,
           self_k1, nbr_k1, b1, mlp_w1, mlp_b1, mlp_w2, mlp_b2):
    zeros = jnp.zeros((NP, D), jnp.float32)
    b0r = b0.reshape(1, 2 * H)
    b1r = b1.reshape(1, 2 * H)
    sk1r = self_k1.reshape(2, H, H)
    nk1r = nbr_k1.reshape(2, H, H)
    w1r = mlp_w1.reshape(2, H, H)
    b1mr = mlp_b1.reshape(1, H)
    b2mr = mlp_b2.reshape(1, C)
    ei4 = edge_index.reshape(2, NW, CPW, CB)

    agg0, degp = _sc_agg(x, ei4, zeros, True, split_edges=True)
    degp = degp.reshape(NW, NP)
    s1a, t1a = _tc_pre1(x, self_k0, b0r, sk1r, nk1r)

    t1, h1b = _tc_mid(agg0, degp, t1a, nbr_k0, b0r, nk1r)

    (agg1,) = _sc_agg(t1, ei4, zeros, False, split_edges=False)
    p3 = _tc_pre2(s1a, h1b, sk1r, b1r, w1r)

    return _tc_post(agg1, degp, p3, b1r, w1r, b1mr, mlp_w2, b2mr)


# speedup vs baseline: 14.6672x; 1.0425x over previous
import jax
import jax.numpy as jnp
from jax import lax
from jax.experimental import pallas as pl
from jax.experimental.pallas import tpu as pltpu
from jax.experimental.pallas import tpu_sc as plsc

N = 10000
NP = 10240
E = 320000
D = 128
H = 256
C = 6

NCORE = 2
NSUB = 16
CB = 125
NW = NCORE * NSUB
CPW = E // (NW * CB)
RPT = NP // NSUB
BM = 1024
GRID = NP // BM
STAGE = 16


def _sc_agg(table, ei4, zeros, with_deg, *, split_edges, stage):
    mesh = plsc.VectorSubcoreMesh(core_axis_name="c", subcore_axis_name="s")
    out_type = [jax.ShapeDtypeStruct((NCORE, NP, D), jnp.float32)]
    if with_deg:
        out_type.append(jax.ShapeDtypeStruct((NW, 1, NP), jnp.float32))
    spw = CPW // stage
    phases = spw if split_edges else 2 * spw
    scratch = [
        pltpu.VMEM_SHARED((NP, D), jnp.float32),
        pltpu.VMEM((stage, CB), jnp.int32),
        pltpu.VMEM((stage, CB), jnp.int32),
        pltpu.VMEM((CB, D), jnp.float32),
        pltpu.VMEM((CB, D), jnp.float32),
        pltpu.SemaphoreType.DMA,
        pltpu.SemaphoreType.DMA,
    ]
    if with_deg:
        scratch.append(pltpu.VMEM((NP,), jnp.float32))

    def body(*refs):
        if with_deg:
            (table_h, ei_h, zeros_h, agg_o, deg_o,
             acc, row_v, col_v, gbuf0, gbuf1, sem0, sem1, deg_v) = refs
        else:
            (table_h, ei_h, zeros_h, agg_o,
             acc, row_v, col_v, gbuf0, gbuf1, sem0, sem1) = refs
        c = lax.axis_index("c")
        s = lax.axis_index("s")
        tbl = table_h.at[c] if table.ndim == 3 else table_h

        pltpu.sync_copy(zeros_h.at[pl.ds(s * RPT, RPT)],
                        acc.at[pl.ds(s * RPT, RPT)])

        if with_deg:
            zeros16 = jnp.zeros((16,), jnp.float32)

            def dzero(i, carry):
                deg_v[pl.ds(i * 16, 16)] = zeros16
                return carry

            lax.fori_loop(0, NP // 16, dzero, 0)

        plsc.subcore_barrier()

        ones16 = jnp.ones((16,), jnp.float32)
        tailmask = lax.iota(jnp.int32, 16) >= 3

        def dupd(j):
            if with_deg:
                for k in range(7):
                    idx = row_v[j, pl.ds(k * 16, 16)]
                    plsc.addupdate_scatter(deg_v, [idx], ones16)
                idx = row_v[j, pl.ds(CB - 16, 16)]
                plsc.addupdate_scatter(deg_v, [idx], ones16, mask=tailmask)

        for p in range(phases):
            if split_edges:
                wrow = c * NSUB + s
                roff = p * stage
            else:
                wrow = 2 * s + p // spw
                roff = (p % spw) * stage
            pltpu.sync_copy(ei_h.at[0, wrow, pl.ds(roff, stage)], row_v)
            pltpu.sync_copy(ei_h.at[1, wrow, pl.ds(roff, stage)], col_v)

            pltpu.async_copy(tbl.at[col_v.at[0]], gbuf0, sem0)

            def pair(i, carry):
                j0 = 2 * i
                pltpu.async_copy(tbl.at[col_v.at[j0 + 1]], gbuf1, sem1)
                pltpu.make_async_copy(
                    tbl.at[col_v.at[j0]], gbuf0, sem0).wait()
                dupd(j0)
                pltpu.sync_copy(gbuf0, acc.at[row_v.at[j0]], add=True)

                @pl.when(i < stage // 2 - 1)
                def _():
                    pltpu.async_copy(tbl.at[col_v.at[j0 + 2]], gbuf0, sem0)

                pltpu.make_async_copy(
                    tbl.at[col_v.at[j0 + 1]], gbuf1, sem1).wait()
                dupd(j0 + 1)
                pltpu.sync_copy(gbuf1, acc.at[row_v.at[j0 + 1]], add=True)
                return carry

            lax.fori_loop(0, stage // 2, pair, 0)

        if with_deg:
            w = c * NSUB + s
            pltpu.sync_copy(deg_v, deg_o.at[w, 0])

        plsc.subcore_barrier()
        pltpu.sync_copy(acc.at[pl.ds(s * RPT, RPT)],
                        agg_o.at[c, pl.ds(s * RPT, RPT)])

    return pl.kernel(
        body, out_type=out_type, mesh=mesh, scratch_types=scratch,
        compiler_params=pltpu.CompilerParams(needs_layout_passes=False),
    )(table, ei4, zeros)


_f32 = jnp.float32


def _dot(a, b):
    return jnp.dot(a, b, preferred_element_type=_f32)


def _wspec(*shape):
    nd = len(shape)
    return pl.BlockSpec(shape, lambda i, nd=nd: (0,) * nd)


def _rspec(*shape):
    return pl.BlockSpec(shape, lambda i: (i,) + (0,) * (len(shape) - 1))


_spec2 = pl.BlockSpec((2, BM, D), lambda i: (0, i, 0))
_dspec = pl.BlockSpec((NW, BM), lambda i: (0, i))


def _tc_pre1(x, sk0, b0r, sk1r, nk1r):
    def body(x_r, sk0_r, b0_r, sk1_r, nk1_r, s1a_r, t1a_r):
        h1a = jnp.maximum(_dot(x_r[...], sk0_r[...]) + b0_r[0, :H], 0.0)
        s1a_r[...] = _dot(h1a, sk1_r[0])
        t1a_r[...] = _dot(h1a, nk1_r[0])

    return pl.pallas_call(
        body,
        grid=(GRID,),
        in_specs=[_rspec(BM, D), _wspec(D, H), _wspec(1, 2 * H),
                  _wspec(2, H, H), _wspec(2, H, H)],
        out_specs=[_rspec(BM, H), _rspec(BM, H)],
        out_shape=[jax.ShapeDtypeStruct((NP, H), _f32),
                   jax.ShapeDtypeStruct((NP, H), _f32)],
    )(x, sk0, b0r, sk1r, nk1r)


def _tc_mid(agg0, degp, t1a, nk0, b0r, nk1r):
    def body(a_r, degp_r, t1a_r, nk0_r, b0_r, nk1_r, t1_r, h1b_r):
        deg = jnp.maximum(jnp.sum(degp_r[...], axis=0), 1.0)
        mean = (a_r[0] + a_r[1]) / deg[:, None]
        h1b = jnp.maximum(_dot(mean, nk0_r[...]) + b0_r[0, H:], 0.0)
        h1b_r[...] = h1b
        t1 = t1a_r[...] + _dot(h1b, nk1_r[1])
        t1_r[0] = t1[:, :D]
        t1_r[1] = t1[:, D:]

    return pl.pallas_call(
        body,
        grid=(GRID,),
        in_specs=[_spec2, _dspec, _rspec(BM, H),
                  _wspec(D, H), _wspec(1, 2 * H), _wspec(2, H, H)],
        out_specs=[_spec2, _rspec(BM, H)],
        out_shape=[jax.ShapeDtypeStruct((2, NP, D), _f32),
                   jax.ShapeDtypeStruct((NP, H), _f32)],
    )(agg0, degp, t1a, nk0, b0r, nk1r)


def _tc_pre2(s1a, h1b, sk1r, b1r, w1r):
    def body(s1a_r, h1b_r, sk1_r, b1_r, w1_r, p3_r):
        s1 = s1a_r[...] + _dot(h1b_r[...], sk1_r[1])
        h2a = jnp.maximum(s1 + b1_r[0, :H], 0.0)
        p3_r[...] = _dot(h2a, w1_r[0])

    return pl.pallas_call(
        body,
        grid=(GRID,),
        in_specs=[_rspec(BM, H), _rspec(BM, H), _wspec(2, H, H),
                  _wspec(1, 2 * H), _wspec(2, H, H)],
        out_specs=_rspec(BM, H),
        out_shape=jax.ShapeDtypeStruct((NP, H), _f32),
    )(s1a, h1b, sk1r, b1r, w1r)


def _tc_post(agg1, degp, p3, b1r, w1r, b1mr, w2, b2mr):
    def body(a_r, degp_r, p3_r, b1_r, w1_r, b1m_r, w2_r, b2m_r, o_r):
        deg = jnp.maximum(jnp.sum(degp_r[...], axis=0), 1.0)
        m = jnp.concatenate([a_r[0], a_r[1]], axis=1) / deg[:, None]
        h2b = jnp.maximum(m + b1_r[0, H:], 0.0)
        h3 = jnp.maximum(p3_r[...] + _dot(h2b, w1_r[1]) + b1m_r[0], 0.0)
        o_r[...] = _dot(h3, w2_r[...]) + b2m_r[0]

    return pl.pallas_call(
        body,
        grid=(GRID,),
        in_specs=[_spec2, _dspec,
                  _rspec(BM, H), _wspec(1, 2 * H), _wspec(2, H, H),
                  _wspec(1, H), _wspec(H, C), _wspec(1, C)],
        out_specs=_rspec(BM, C),
        out_shape=jax.ShapeDtypeStruct((N, C), _f32),
    )(agg1, degp, p3, b1r, w1r, b1mr, w2, b2mr)


def kernel(x, edge_index, edge_weight, self_k0, nbr_k0, b0,
           self_k1, nbr_k1, b1, mlp_w1, mlp_b1, mlp_w2, mlp_b2):
    zeros = jnp.zeros((NP, D), jnp.float32)
    b0r = b0.reshape(1, 2 * H)
    b1r = b1.reshape(1, 2 * H)
    sk1r = self_k1.reshape(2, H, H)
    nk1r = nbr_k1.reshape(2, H, H)
    w1r = mlp_w1.reshape(2, H, H)
    b1mr = mlp_b1.reshape(1, H)
    b2mr = mlp_b2.reshape(1, C)
    ei4 = edge_index.reshape(2, NW, CPW, CB)

    agg0, degp = _sc_agg(x, ei4, zeros, True, split_edges=True, stage=16)
    degp = degp.reshape(NW, NP)
    s1a, t1a = _tc_pre1(x, self_k0, b0r, sk1r, nk1r)

    t1, h1b = _tc_mid(agg0, degp, t1a, nbr_k0, b0r, nk1r)

    (agg1,) = _sc_agg(t1, ei4, zeros, False, split_edges=False, stage=40)

    p3 = _tc_pre2(s1a, h1b, sk1r, b1r, w1r)

    return _tc_post(agg1, degp, p3, b1r, w1r, b1mr, mlp_w2, b2mr)


# speedup vs baseline: 14.7585x; 1.0062x over previous
import jax
import jax.numpy as jnp
from jax import lax
from jax.experimental import pallas as pl
from jax.experimental.pallas import tpu as pltpu
from jax.experimental.pallas import tpu_sc as plsc

N = 10000
NP = 10240
E = 320000
D = 128
H = 256
C = 6

NCORE = 2
NSUB = 16
CB = 125
NW = NCORE * NSUB
CPW = E // (NW * CB)
RPT = NP // NSUB
BM = 1024
GRID = NP // BM
STAGE = 16


def _sc_agg(table, ei4, zeros, with_deg, *, split_edges, stage):
    mesh = plsc.VectorSubcoreMesh(core_axis_name="c", subcore_axis_name="s")
    out_type = [jax.ShapeDtypeStruct((NCORE, NP, D), jnp.float32)]
    if with_deg:
        out_type.append(jax.ShapeDtypeStruct((NW, 1, NP), jnp.float32))
    spw = CPW // stage
    phases = spw if split_edges else 2 * spw
    scratch = [
        pltpu.VMEM_SHARED((NP, D), jnp.float32),
        pltpu.VMEM((stage, CB), jnp.int32),
        pltpu.VMEM((stage, CB), jnp.int32),
        pltpu.VMEM((CB, D), jnp.float32),
        pltpu.VMEM((CB, D), jnp.float32),
        pltpu.SemaphoreType.DMA,
        pltpu.SemaphoreType.DMA,
    ]
    if with_deg:
        scratch.append(pltpu.VMEM((NP,), jnp.float32))

    def body(*refs):
        if with_deg:
            (table_h, ei_h, zeros_h, agg_o, deg_o,
             acc, row_v, col_v, gbuf0, gbuf1, sem0, sem1, deg_v) = refs
        else:
            (table_h, ei_h, zeros_h, agg_o,
             acc, row_v, col_v, gbuf0, gbuf1, sem0, sem1) = refs
        c = lax.axis_index("c")
        s = lax.axis_index("s")
        tbl = table_h.at[c] if table.ndim == 3 else table_h

        pltpu.sync_copy(zeros_h.at[pl.ds(s * RPT, RPT)],
                        acc.at[pl.ds(s * RPT, RPT)])

        if with_deg:
            zeros16 = jnp.zeros((16,), jnp.float32)

            def dzero(i, carry):
                deg_v[pl.ds(i * 16, 16)] = zeros16
                return carry

            lax.fori_loop(0, NP // 16, dzero, 0)

        plsc.subcore_barrier()

        ones16 = jnp.ones((16,), jnp.float32)
        tailmask = lax.iota(jnp.int32, 16) >= 3

        def dupd(j):
            if with_deg:
                for k in range(7):
                    idx = row_v[j, pl.ds(k * 16, 16)]
                    plsc.addupdate_scatter(deg_v, [idx], ones16)
                idx = row_v[j, pl.ds(CB - 16, 16)]
                plsc.addupdate_scatter(deg_v, [idx], ones16, mask=tailmask)

        for p in range(phases):
            if split_edges:
                wrow = c * NSUB + s
                roff = p * stage
            else:
                wrow = 2 * s + p // spw
                roff = (p % spw) * stage
            pltpu.sync_copy(ei_h.at[0, wrow, pl.ds(roff, stage)], row_v)
            pltpu.sync_copy(ei_h.at[1, wrow, pl.ds(roff, stage)], col_v)

            pltpu.async_copy(tbl.at[col_v.at[0]], gbuf0, sem0)

            def pair(i, carry):
                j0 = 2 * i
                pltpu.async_copy(tbl.at[col_v.at[j0 + 1]], gbuf1, sem1)
                pltpu.make_async_copy(
                    tbl.at[col_v.at[j0]], gbuf0, sem0).wait()
                dupd(j0)
                pltpu.sync_copy(gbuf0, acc.at[row_v.at[j0]], add=True)

                @pl.when(i < stage // 2 - 1)
                def _():
                    pltpu.async_copy(tbl.at[col_v.at[j0 + 2]], gbuf0, sem0)

                pltpu.make_async_copy(
                    tbl.at[col_v.at[j0 + 1]], gbuf1, sem1).wait()
                dupd(j0 + 1)
                pltpu.sync_copy(gbuf1, acc.at[row_v.at[j0 + 1]], add=True)
                return carry

            lax.fori_loop(0, stage // 2, pair, 0)

        if with_deg:
            w = c * NSUB + s
            pltpu.sync_copy(deg_v, deg_o.at[w, 0])

        plsc.subcore_barrier()
        pltpu.sync_copy(acc.at[pl.ds(s * RPT, RPT)],
                        agg_o.at[c, pl.ds(s * RPT, RPT)])

    return pl.kernel(
        body, out_type=out_type, mesh=mesh, scratch_types=scratch,
        compiler_params=pltpu.CompilerParams(needs_layout_passes=False),
    )(table, ei4, zeros)


_f32 = jnp.float32


def _dot(a, b):
    return jnp.dot(a, b, preferred_element_type=_f32)


def _wspec(*shape):
    nd = len(shape)
    return pl.BlockSpec(shape, lambda i, nd=nd: (0,) * nd)


def _rspec(*shape):
    return pl.BlockSpec(shape, lambda i: (i,) + (0,) * (len(shape) - 1))


_spec2 = pl.BlockSpec((2, BM, D), lambda i: (0, i, 0))
_dspec = pl.BlockSpec((NW, 1, BM), lambda i: (0, 0, i))


def _tc_pre1(x, sk0, b0r, sk1r, nk1r):
    def body(x_r, sk0_r, b0_r, sk1_r, nk1_r, s1a_r, t1a_r):
        h1a = jnp.maximum(_dot(x_r[...], sk0_r[...]) + b0_r[0, :H], 0.0)
        s1a_r[...] = _dot(h1a, sk1_r[0])
        t1a_r[...] = _dot(h1a, nk1_r[0])

    return pl.pallas_call(
        body,
        grid=(GRID,),
        in_specs=[_rspec(BM, D), _wspec(D, H), _wspec(1, 2 * H),
                  _wspec(2, H, H), _wspec(2, H, H)],
        out_specs=[_rspec(BM, H), _rspec(BM, H)],
        out_shape=[jax.ShapeDtypeStruct((NP, H), _f32),
                   jax.ShapeDtypeStruct((NP, H), _f32)],
    )(x, sk0, b0r, sk1r, nk1r)


def _tc_mid(agg0, degp, t1a, nk0, b0r, nk1r):
    def body(a_r, degp_r, t1a_r, nk0_r, b0_r, nk1_r, t1_r, h1b_r):
        deg = jnp.maximum(jnp.sum(degp_r[...], axis=(0, 1)), 1.0)
        mean = (a_r[0] + a_r[1]) / deg[:, None]
        h1b = jnp.maximum(_dot(mean, nk0_r[...]) + b0_r[0, H:], 0.0)
        h1b_r[...] = h1b
        t1 = t1a_r[...] + _dot(h1b, nk1_r[1])
        t1_r[0] = t1[:, :D]
        t1_r[1] = t1[:, D:]

    return pl.pallas_call(
        body,
        grid=(GRID,),
        in_specs=[_spec2, _dspec, _rspec(BM, H),
                  _wspec(D, H), _wspec(1, 2 * H), _wspec(2, H, H)],
        out_specs=[_spec2, _rspec(BM, H)],
        out_shape=[jax.ShapeDtypeStruct((2, NP, D), _f32),
                   jax.ShapeDtypeStruct((NP, H), _f32)],
    )(agg0, degp, t1a, nk0, b0r, nk1r)


def _tc_pre2(s1a, h1b, sk1r, b1r, w1r):
    def body(s1a_r, h1b_r, sk1_r, b1_r, w1_r, p3_r):
        s1 = s1a_r[...] + _dot(h1b_r[...], sk1_r[1])
        h2a = jnp.maximum(s1 + b1_r[0, :H], 0.0)
        p3_r[...] = _dot(h2a, w1_r[0])

    return pl.pallas_call(
        body,
        grid=(GRID,),
        in_specs=[_rspec(BM, H), _rspec(BM, H), _wspec(2, H, H),
                  _wspec(1, 2 * H), _wspec(2, H, H)],
        out_specs=_rspec(BM, H),
        out_shape=jax.ShapeDtypeStruct((NP, H), _f32),
    )(s1a, h1b, sk1r, b1r, w1r)


def _tc_post(agg1, degp, p3, b1r, w1r, b1mr, w2, b2mr):
    def body(a_r, degp_r, p3_r, b1_r, w1_r, b1m_r, w2_r, b2m_r, o_r):
        deg = jnp.maximum(jnp.sum(degp_r[...], axis=(0, 1)), 1.0)
        m = jnp.concatenate([a_r[0], a_r[1]], axis=1) / deg[:, None]
        h2b = jnp.maximum(m + b1_r[0, H:], 0.0)
        h3 = jnp.maximum(p3_r[...] + _dot(h2b, w1_r[1]) + b1m_r[0], 0.0)
        o_r[...] = _dot(h3, w2_r[...]) + b2m_r[0]

    return pl.pallas_call(
        body,
        grid=(GRID,),
        in_specs=[_spec2, _dspec,
                  _rspec(BM, H), _wspec(1, 2 * H), _wspec(2, H, H),
                  _wspec(1, H), _wspec(H, C), _wspec(1, C)],
        out_specs=_rspec(BM, C),
        out_shape=jax.ShapeDtypeStruct((N, C), _f32),
    )(agg1, degp, p3, b1r, w1r, b1mr, w2, b2mr)


def kernel(x, edge_index, edge_weight, self_k0, nbr_k0, b0,
           self_k1, nbr_k1, b1, mlp_w1, mlp_b1, mlp_w2, mlp_b2):
    zeros = jnp.zeros((NP, D), jnp.float32)
    b0r = b0.reshape(1, 2 * H)
    b1r = b1.reshape(1, 2 * H)
    sk1r = self_k1.reshape(2, H, H)
    nk1r = nbr_k1.reshape(2, H, H)
    w1r = mlp_w1.reshape(2, H, H)
    b1mr = mlp_b1.reshape(1, H)
    b2mr = mlp_b2.reshape(1, C)
    ei4 = edge_index.reshape(2, NW, CPW, CB)

    agg0, degp = _sc_agg(x, ei4, zeros, True, split_edges=True, stage=16)
    s1a, t1a = _tc_pre1(x, self_k0, b0r, sk1r, nk1r)

    t1, h1b = _tc_mid(agg0, degp, t1a, nbr_k0, b0r, nk1r)

    (agg1,) = _sc_agg(t1, ei4, zeros, False, split_edges=False, stage=40)

    p3 = _tc_pre2(s1a, h1b, sk1r, b1r, w1r)

    return _tc_post(agg1, degp, p3, b1r, w1r, b1mr, mlp_w2, b2mr)


# speedup vs baseline: 15.0625x; 1.0206x over previous
import jax
import jax.numpy as jnp
from jax import lax
from jax.experimental import pallas as pl
from jax.experimental.pallas import tpu as pltpu
from jax.experimental.pallas import tpu_sc as plsc

N = 10000
NP = 10240
E = 320000
D = 128
H = 256
C = 6

NCORE = 2
NSUB = 16
CB = 125
NW = NCORE * NSUB
CPW = E // (NW * CB)
RPT = NP // NSUB
BM = 1024
GRID = NP // BM
STAGE = 16


def _sc_agg(table, ei4, zeros, with_deg, *, split_edges, stage):
    mesh = plsc.VectorSubcoreMesh(core_axis_name="c", subcore_axis_name="s")
    out_type = [jax.ShapeDtypeStruct((NCORE, NP, D), jnp.float32)]
    if with_deg:
        out_type.append(jax.ShapeDtypeStruct((NW, 1, NP), jnp.float32))
    spw = CPW // stage
    phases = spw if split_edges else 2 * spw
    scratch = [
        pltpu.VMEM_SHARED((NP, D), jnp.float32),
        pltpu.VMEM((stage, CB), jnp.int32),
        pltpu.VMEM((stage, CB), jnp.int32),
        pltpu.VMEM((CB, D), jnp.float32),
        pltpu.VMEM((CB, D), jnp.float32),
        pltpu.SemaphoreType.DMA,
        pltpu.SemaphoreType.DMA,
    ]
    if with_deg:
        scratch.append(pltpu.VMEM((NP,), jnp.float32))

    def body(*refs):
        if with_deg:
            (table_h, ei_h, zeros_h, agg_o, deg_o,
             acc, row_v, col_v, gbuf0, gbuf1, sem0, sem1, deg_v) = refs
        else:
            (table_h, ei_h, zeros_h, agg_o,
             acc, row_v, col_v, gbuf0, gbuf1, sem0, sem1) = refs
        c = lax.axis_index("c")
        s = lax.axis_index("s")
        tbl = table_h.at[c] if table.ndim == 3 else table_h

        pltpu.sync_copy(zeros_h.at[pl.ds(s * RPT, RPT)],
                        acc.at[pl.ds(s * RPT, RPT)])

        if with_deg:
            zeros16 = jnp.zeros((16,), jnp.float32)

            def dzero(i, carry):
                deg_v[pl.ds(i * 16, 16)] = zeros16
                return carry

            lax.fori_loop(0, NP // 16, dzero, 0)

        plsc.subcore_barrier()

        ones16 = jnp.ones((16,), jnp.float32)
        tailmask = lax.iota(jnp.int32, 16) >= 3

        def dupd(j):
            if with_deg:
                for k in range(7):
                    idx = row_v[j, pl.ds(k * 16, 16)]
                    plsc.addupdate_scatter(deg_v, [idx], ones16)
                idx = row_v[j, pl.ds(CB - 16, 16)]
                plsc.addupdate_scatter(deg_v, [idx], ones16, mask=tailmask)

        for p in range(phases):
            if split_edges:
                wrow = c * NSUB + s
                roff = p * stage
            else:
                wrow = 2 * s + p // spw
                roff = (p % spw) * stage
            pltpu.sync_copy(ei_h.at[0, wrow, pl.ds(roff, stage)], row_v)
            pltpu.sync_copy(ei_h.at[1, wrow, pl.ds(roff, stage)], col_v)

            pltpu.async_copy(tbl.at[col_v.at[0]], gbuf0, sem0)

            def pair(i, carry):
                j0 = 2 * i
                pltpu.async_copy(tbl.at[col_v.at[j0 + 1]], gbuf1, sem1)
                pltpu.make_async_copy(
                    tbl.at[col_v.at[j0]], gbuf0, sem0).wait()
                dupd(j0)
                pltpu.sync_copy(gbuf0, acc.at[row_v.at[j0]], add=True)

                @pl.when(i < stage // 2 - 1)
                def _():
                    pltpu.async_copy(tbl.at[col_v.at[j0 + 2]], gbuf0, sem0)

                pltpu.make_async_copy(
                    tbl.at[col_v.at[j0 + 1]], gbuf1, sem1).wait()
                dupd(j0 + 1)
                pltpu.sync_copy(gbuf1, acc.at[row_v.at[j0 + 1]], add=True)
                return carry

            lax.fori_loop(0, stage // 2, pair, 0)

        if with_deg:
            w = c * NSUB + s
            pltpu.sync_copy(deg_v, deg_o.at[w, 0])

        plsc.subcore_barrier()
        pltpu.sync_copy(acc.at[pl.ds(s * RPT, RPT)],
                        agg_o.at[c, pl.ds(s * RPT, RPT)])

    return pl.kernel(
        body, out_type=out_type, mesh=mesh, scratch_types=scratch,
        compiler_params=pltpu.CompilerParams(needs_layout_passes=False),
    )(table, ei4, zeros)


_f32 = jnp.float32


def _dot(a, b):
    return jnp.dot(a, b, preferred_element_type=_f32)


def _wspec(*shape):
    nd = len(shape)
    return pl.BlockSpec(shape, lambda i, nd=nd: (0,) * nd)


def _rspec(*shape):
    return pl.BlockSpec(shape, lambda i: (i,) + (0,) * (len(shape) - 1))


_spec2 = pl.BlockSpec((2, BM, D), lambda i: (0, i, 0))
_dspec = pl.BlockSpec((NW, 1, BM), lambda i: (0, 0, i))


def _tc_pre1(x, sk0, b0r, sk1r, nk1r):
    def body(x_r, sk0_r, b0_r, sk1_r, nk1_r, s1a_r, t1a_r):
        h1a = jnp.maximum(_dot(x_r[...], sk0_r[...]) + b0_r[0, :H], 0.0)
        s1a_r[...] = _dot(h1a, sk1_r[0])
        t1a_r[...] = _dot(h1a, nk1_r[0])

    return pl.pallas_call(
        body,
        grid=(GRID,),
        in_specs=[_rspec(BM, D), _wspec(D, H), _wspec(1, 2 * H),
                  _wspec(2, H, H), _wspec(2, H, H)],
        out_specs=[_rspec(BM, H), _rspec(BM, H)],
        out_shape=[jax.ShapeDtypeStruct((NP, H), _f32),
                   jax.ShapeDtypeStruct((NP, H), _f32)],
    )(x, sk0, b0r, sk1r, nk1r)


def _tc_mid(agg0, degp, t1a, nk0, b0r, nk1r):
    def body(a_r, degp_r, t1a_r, nk0_r, b0_r, nk1_r, t1_r, h1b_r):
        deg = jnp.maximum(jnp.sum(degp_r[...].reshape(NW, BM), axis=0), 1.0)
        mean = (a_r[0] + a_r[1]) / deg[:, None]
        h1b = jnp.maximum(_dot(mean, nk0_r[...]) + b0_r[0, H:], 0.0)
        h1b_r[...] = h1b
        t1 = t1a_r[...] + _dot(h1b, nk1_r[1])
        t1_r[0] = t1[:, :D]
        t1_r[1] = t1[:, D:]

    return pl.pallas_call(
        body,
        grid=(GRID,),
        in_specs=[_spec2, _dspec, _rspec(BM, H),
                  _wspec(D, H), _wspec(1, 2 * H), _wspec(2, H, H)],
        out_specs=[_spec2, _rspec(BM, H)],
        out_shape=[jax.ShapeDtypeStruct((2, NP, D), _f32),
                   jax.ShapeDtypeStruct((NP, H), _f32)],
    )(agg0, degp, t1a, nk0, b0r, nk1r)


def _tc_pre2(s1a, h1b, sk1r, b1r, w1r):
    def body(s1a_r, h1b_r, sk1_r, b1_r, w1_r, p3_r):
        s1 = s1a_r[...] + _dot(h1b_r[...], sk1_r[1])
        h2a = jnp.maximum(s1 + b1_r[0, :H], 0.0)
        p3_r[...] = _dot(h2a, w1_r[0])

    return pl.pallas_call(
        body,
        grid=(GRID,),
        in_specs=[_rspec(BM, H), _rspec(BM, H), _wspec(2, H, H),
                  _wspec(1, 2 * H), _wspec(2, H, H)],
        out_specs=_rspec(BM, H),
        out_shape=jax.ShapeDtypeStruct((NP, H), _f32),
    )(s1a, h1b, sk1r, b1r, w1r)


def _tc_post(agg1, degp, p3, b1r, w1r, b1mr, w2, b2mr):
    def body(a_r, degp_r, p3_r, b1_r, w1_r, b1m_r, w2_r, b2m_r, o_r):
        deg = jnp.maximum(jnp.sum(degp_r[...].reshape(NW, BM), axis=0), 1.0)
        m = jnp.concatenate([a_r[0], a_r[1]], axis=1) / deg[:, None]
        h2b = jnp.maximum(m + b1_r[0, H:], 0.0)
        h3 = jnp.maximum(p3_r[...] + _dot(h2b, w1_r[1]) + b1m_r[0], 0.0)
        o_r[...] = _dot(h3, w2_r[...]) + b2m_r[0]

    return pl.pallas_call(
        body,
        grid=(GRID,),
        in_specs=[_spec2, _dspec,
                  _rspec(BM, H), _wspec(1, 2 * H), _wspec(2, H, H),
                  _wspec(1, H), _wspec(H, C), _wspec(1, C)],
        out_specs=_rspec(BM, C),
        out_shape=jax.ShapeDtypeStruct((N, C), _f32),
    )(agg1, degp, p3, b1r, w1r, b1mr, w2, b2mr)


def kernel(x, edge_index, edge_weight, self_k0, nbr_k0, b0,
           self_k1, nbr_k1, b1, mlp_w1, mlp_b1, mlp_w2, mlp_b2):
    zeros = jnp.zeros((NP, D), jnp.float32)
    b0r = b0.reshape(1, 2 * H)
    b1r = b1.reshape(1, 2 * H)
    sk1r = self_k1.reshape(2, H, H)
    nk1r = nbr_k1.reshape(2, H, H)
    w1r = mlp_w1.reshape(2, H, H)
    b1mr = mlp_b1.reshape(1, H)
    b2mr = mlp_b2.reshape(1, C)
    ei4 = edge_index.reshape(2, NW, CPW, CB)

    agg0, degp = _sc_agg(x, ei4, zeros, True, split_edges=True, stage=16)
    s1a, t1a = _tc_pre1(x, self_k0, b0r, sk1r, nk1r)

    t1, h1b = _tc_mid(agg0, degp, t1a, nbr_k0, b0r, nk1r)

    (agg1,) = _sc_agg(t1, ei4, zeros, False, split_edges=False, stage=40)

    p3 = _tc_pre2(s1a, h1b, sk1r, b1r, w1r)

    return _tc_post(agg1, degp, p3, b1r, w1r, b1mr, mlp_w2, b2mr)
